# baseline (speedup 1.0000x reference)
import functools

import jax
import jax.numpy as jnp
from jax import lax
from jax.experimental import pallas as pl
from jax.experimental.pallas import tpu as pltpu
from jax.experimental.pallas import tpu_sc as plsc

N = 100000
E = 3200000
G = 128
NC, NS = 2, 16
WIN = 128
NWIN = E // WIN
IDXS = 3
ROWS = 2
CCH = 6256
CREM = N - 15 * CCH

ROWS_BLK = 4000
NBLK = N // ROWS_BLK


def _sc_agg(hs, ei, zrows, zflat, with_cnt):
    nh = len(hs)
    per_core = nh // 2
    kb = 4 if with_cnt else 5
    nblock = NWIN // kb
    bpt = nblock // NS
    brem = nblock - bpt * NS
    mesh = plsc.VectorSubcoreMesh(core_axis_name="c", subcore_axis_name="s")
    out_type = [jax.ShapeDtypeStruct((N, 16), jnp.float32)] * nh
    scratch = [
        pltpu.VMEM((IDXS, kb, WIN), jnp.int32),
        pltpu.VMEM((IDXS, kb, WIN), jnp.int32),
        pltpu.VMEM((ROWS, kb, WIN, 16), jnp.float32),
        pltpu.VMEM_SHARED((N, 16), jnp.float32),
        pltpu.SemaphoreType.DMA,
        pltpu.SemaphoreType.DMA((kb,)),
        pltpu.SemaphoreType.DMA,
    ]
    if with_cnt:
        out_type.append(jax.ShapeDtypeStruct((N,), jnp.float32))
        scratch += [
            pltpu.VMEM((WIN,), jnp.float32),
            pltpu.VMEM_SHARED((N,), jnp.float32),
        ]

    def body(*refs):
        h_hbms = refs[:nh]
        ei_hbm, zrows_hbm, zflat_hbm = refs[nh:nh + 3]
        out_hbms = refs[nh + 3:nh + 3 + nh]
        rest = refs[nh + 3 + nh:]
        if with_cnt:
            (cnt_hbm, src_v, dst_v, rows_v, agg_sh, isem, gsem, ssem,
             ones_v, cnt_sh) = rest
        else:
            src_v, dst_v, rows_v, agg_sh, isem, gsem, ssem = rest
        c = lax.axis_index("c")
        s = lax.axis_index("s")

        def chunked(fn):
            @pl.when(s < 15)
            def _():
                fn(s * CCH, CCH)

            @pl.when(s == 15)
            def _():
                fn(15 * CCH, CREM)

        def zero_agg(off, sz):
            pltpu.sync_copy(zrows_hbm.at[pl.ds(off, sz)],
                            agg_sh.at[pl.ds(off, sz)])
        chunked(zero_agg)
        if with_cnt:
            for i in range(WIN // 16):
                ones_v[pl.ds(i * 16, 16)] = jnp.ones((16,), jnp.float32)

            @pl.when(c == 0)
            def _():
                def zero_cnt(off, sz):
                    pltpu.sync_copy(zflat_hbm.at[pl.ds(off, sz)],
                                    cnt_sh.at[pl.ds(off, sz)])
                chunked(zero_cnt)
        plsc.subcore_barrier()

        base = s * bpt + jnp.minimum(s, brem)
        nblk = jnp.where(s < brem, bpt + 1, bpt)

        def sweep(h_hbm, count_too):
            def issue_idx(g, islot):
                off = (base + g) * kb
                pltpu.async_copy(ei_hbm.at[0, pl.ds(off, kb)],
                                 src_v.at[islot], isem)
                pltpu.async_copy(ei_hbm.at[1, pl.ds(off, kb)],
                                 dst_v.at[islot], isem)

            def drain_block_scatters(rslot):
                for j in range(kb):
                    pltpu.make_async_copy(
                        zrows_hbm.at[pl.ds(0, WIN)], rows_v.at[rslot, j],
                        ssem).wait()
                    if count_too:
                        pltpu.make_async_copy(
                            zflat_hbm.at[pl.ds(0, WIN)], ones_v,
                            ssem).wait()

            issue_idx(0, 0)

            def step(g, carry):
                islot = lax.rem(g, IDXS)
                rslot = lax.rem(g, ROWS)
                pltpu.make_async_copy(ei_hbm.at[0, pl.ds(0, kb)],
                                      src_v.at[islot], isem).wait()
                pltpu.make_async_copy(ei_hbm.at[1, pl.ds(0, kb)],
                                      dst_v.at[islot], isem).wait()

                @pl.when(g >= ROWS)
                def _():
                    drain_block_scatters(rslot)

                @pl.when(g + 1 < nblk)
                def _():
                    issue_idx(g + 1, lax.rem(g + 1, IDXS))

                gds = [pltpu.async_copy(h_hbm.at[src_v.at[islot, j]],
                                        rows_v.at[rslot, j], gsem.at[j])
                       for j in range(kb)]
                for j in range(kb):
                    gds[j].wait()
                    pltpu.async_copy(rows_v.at[rslot, j],
                                     agg_sh.at[dst_v.at[islot, j]], ssem,
                                     add=True)
                    if count_too:
                        pltpu.async_copy(ones_v,
                                         cnt_sh.at[dst_v.at[islot, j]],
                                         ssem, add=True)
                return carry

            lax.fori_loop(0, nblk, step, 0)
            for r in range(ROWS):
                drain_block_scatters(r)

        def phase(pairs, core_cnt):
            for k, (h_hbm, out_hbm) in enumerate(pairs):
                if k > 0:
                    chunked(zero_agg)
                    plsc.subcore_barrier()
                do_cnt = core_cnt and k == 0
                sweep(h_hbm, do_cnt)
                plsc.subcore_barrier()

                def write(off, sz):
                    rows = pl.ds(off, sz)
                    pltpu.sync_copy(agg_sh.at[rows], out_hbm.at[rows])
                    if do_cnt:
                        pltpu.sync_copy(cnt_sh.at[rows], cnt_hbm.at[rows])
                chunked(write)

        @pl.when(c == 0)
        def _():
            phase(list(zip(h_hbms[:per_core], out_hbms[:per_core])),
                  with_cnt)

        @pl.when(c == 1)
        def _():
            phase(list(zip(h_hbms[per_core:], out_hbms[per_core:])),
                  False)

    fn = pl.kernel(
        body, out_type=out_type, mesh=mesh, scratch_types=scratch,
        compiler_params=pltpu.CompilerParams(use_tc_tiling_on_sc=False))
    return fn(*hs, ei, zrows, zflat)


def _pre_body(x_ref, W_ref, b_ref, ha_ref, hb_ref):
    h = jnp.maximum(jnp.dot(x_ref[...], W_ref[...],
                            preferred_element_type=jnp.float32)
                    + b_ref[...][None, :], 0.0)
    ha_ref[...] = h[:, :16]
    hb_ref[...] = h[:, 16:]


def _pre(x, W_pre, b_pre):
    return pl.pallas_call(
        _pre_body,
        grid=(NBLK,),
        in_specs=[
            pl.BlockSpec((ROWS_BLK, 5), lambda i: (i, 0)),
            pl.BlockSpec((5, 32), lambda i: (0, 0)),
            pl.BlockSpec((32,), lambda i: (0,)),
        ],
        out_specs=[
            pl.BlockSpec((ROWS_BLK, 16), lambda i: (i, 0)),
            pl.BlockSpec((ROWS_BLK, 16), lambda i: (i, 0)),
        ],
        out_shape=[jax.ShapeDtypeStruct((N, 16), jnp.float32)] * 2,
    )(x, W_pre, b_pre)


def _sage_update(agg, cnt2d, h, Wl, bl, Wr):
    recip = 1.0 / jnp.maximum(cnt2d, 1.0)
    out = (jnp.dot(agg * recip, Wl,
                   preferred_element_type=jnp.float32)
           + bl[None, :]
           + jnp.dot(h, Wr, preferred_element_type=jnp.float32))
    norm = jnp.sqrt(jnp.sum(out * out, axis=-1, keepdims=True))
    return jnp.maximum(out / jnp.maximum(norm, 1e-12), 0.0)


def _mid_body(aggL_ref, aggR_ref, cnt_ref, ha_ref, hb_ref,
              Wl_ref, bl_ref, Wr_ref, *out_refs):
    agg = jnp.concatenate([aggL_ref[...], aggR_ref[...]], axis=1)
    h = jnp.concatenate([ha_ref[...], hb_ref[...]], axis=1)
    h1 = _sage_update(agg, cnt_ref[...], h, Wl_ref[...], bl_ref[...],
                      Wr_ref[...])
    for k in range(4):
        out_refs[k][...] = h1[:, 16 * k:16 * (k + 1)]


def _mid(aggL, aggR, cnt, ha, hb, Wl1, bl1, Wr1):
    blk16 = pl.BlockSpec((ROWS_BLK, 16), lambda i: (i, 0))
    return pl.pallas_call(
        _mid_body,
        grid=(NBLK,),
        in_specs=[
            blk16, blk16,
            pl.BlockSpec((ROWS_BLK, 1), lambda i: (i, 0)),
            blk16, blk16,
            pl.BlockSpec((32, 64), lambda i: (0, 0)),
            pl.BlockSpec((64,), lambda i: (0,)),
            pl.BlockSpec((32, 64), lambda i: (0, 0)),
        ],
        out_specs=[blk16] * 4,
        out_shape=[jax.ShapeDtypeStruct((N, 16), jnp.float32)] * 4,
    )(aggL, aggR, cnt, ha, hb, Wl1, bl1, Wr1)


def _final_body(aggs_and_more, s_acc, c_acc):
    (aggA, aggB, aggC, aggD, cnt_ref, hA, hB, hC, hD,
     Wl_ref, bl_ref, Wr_ref, batch_ref,
     Wp1_ref, bp1_ref, Wp2_ref, bp2_ref, Wo_ref, bo_ref, out_ref) = \
        aggs_and_more
    i = pl.program_id(0)
    agg = jnp.concatenate([aggA[...], aggB[...], aggC[...], aggD[...]],
                          axis=1)
    h = jnp.concatenate([hA[...], hB[...], hC[...], hD[...]], axis=1)
    h2 = _sage_update(agg, cnt_ref[...], h, Wl_ref[...], bl_ref[...],
                      Wr_ref[...])
    onehot = (batch_ref[...]
              == lax.broadcasted_iota(jnp.int32, (ROWS_BLK, G), 1)
              ).astype(jnp.float32)
    contrib = lax.dot_general(onehot, h2, (((0,), (0,)), ((), ())),
                              preferred_element_type=jnp.float32,
                              precision=lax.Precision.HIGHEST)
    csum = jnp.sum(onehot, axis=0)

    @pl.when(i == 0)
    def _():
        s_acc[...] = contrib
        c_acc[...] = csum[None, :]

    @pl.when(i > 0)
    def _():
        s_acc[...] += contrib
        c_acc[...] += csum[None, :]

    @pl.when(i == NBLK - 1)
    def _():
        cvec = jnp.maximum(c_acc[...][0, :], 1.0)
        g = s_acc[...] / cvec[:, None]
        g = jnp.maximum(jnp.dot(g, Wp1_ref[...],
                                preferred_element_type=jnp.float32)
                        + bp1_ref[...][None, :], 0.0)
        g = jnp.maximum(jnp.dot(g, Wp2_ref[...],
                                preferred_element_type=jnp.float32)
                        + bp2_ref[...][None, :], 0.0)
        out_ref[...] = (jnp.dot(g, Wo_ref[...],
                                preferred_element_type=jnp.float32)
                        + bo_ref[...][None, :])


def _final(aggA, aggB, aggC, aggD, cnt, hA, hB, hC, hD, Wl2, bl2, Wr2,
           batch, Wp1, bp1, Wp2, bp2, Wo, bo):
    blk16 = pl.BlockSpec((ROWS_BLK, 16), lambda i: (i, 0))
    full = lambda *shape: pl.BlockSpec(shape, lambda i: (0,) * len(shape))
    return pl.pallas_call(
        lambda *refs: _final_body(refs[:-2], refs[-2], refs[-1]),
        grid=(NBLK,),
        in_specs=[
            blk16, blk16, blk16, blk16,
            pl.BlockSpec((ROWS_BLK, 1), lambda i: (i, 0)),
            blk16, blk16, blk16, blk16,
            full(64, 64), full(64), full(64, 64),
            pl.BlockSpec((ROWS_BLK, 1), lambda i: (i, 0)),
            full(64, 64), full(64), full(64, 16), full(16),
            full(16, 1), full(1),
        ],
        out_specs=full(G, 1),
        out_shape=jax.ShapeDtypeStruct((G, 1), jnp.float32),
        scratch_shapes=[
            pltpu.VMEM((G, 64), jnp.float32),
            pltpu.VMEM((1, G), jnp.float32),
        ],
    )(aggA, aggB, aggC, aggD, cnt, hA, hB, hC, hD, Wl2, bl2, Wr2,
      batch, Wp1, bp1, Wp2, bp2, Wo, bo)


def kernel(x, edge_index, batch, W_pre, b_pre, Wl1, bl1, Wr1,
           Wl2, bl2, Wr2, Wp1, bp1, Wp2, bp2, Wo, bo):
    zrows = jnp.zeros((N, 16), jnp.float32)
    zflat = jnp.zeros((N,), jnp.float32)
    ei3 = edge_index.reshape(2, NWIN, WIN)

    h0a, h0b = _pre(x, W_pre, b_pre)
    agg1L, agg1R, cnt = _sc_agg((h0a, h0b), ei3, zrows, zflat,
                                with_cnt=True)
    cnt2d = cnt.reshape(N, 1)
    h1a, h1b, h1c, h1d = _mid(agg1L, agg1R, cnt2d, h0a, h0b, Wl1, bl1, Wr1)
    agg2A, agg2B, agg2C, agg2D = _sc_agg((h1a, h1b, h1c, h1d), ei3,
                                         zrows, zflat, with_cnt=False)
    out = _final(agg2A, agg2B, agg2C, agg2D, cnt2d, h1a, h1b, h1c, h1d,
                 Wl2, bl2, Wr2, batch.reshape(N, 1), Wp1, bp1, Wp2, bp2,
                 Wo, bo)
    return jnp.squeeze(out, -1)


# speedup vs baseline: 1.0438x; 1.0438x over previous
import functools

import jax
import jax.numpy as jnp
from jax import lax
from jax.experimental import pallas as pl
from jax.experimental.pallas import tpu as pltpu
from jax.experimental.pallas import tpu_sc as plsc

N = 100000
E = 3200000
G = 128
NC, NS = 2, 16
WIN = 128
NWIN = E // WIN
IDXS = 3
ROWS = 2
CCH = 6256
CREM = N - 15 * CCH

ROWS_BLK = 4000
NBLK = N // ROWS_BLK


def _sc_agg(hs, ei, zrows, zflat, with_cnt):
    nh = len(hs)
    per_core = nh // 2
    kb = 4 if with_cnt else 5
    nblock = NWIN // kb
    bpt = nblock // NS
    brem = nblock - bpt * NS
    mesh = plsc.VectorSubcoreMesh(core_axis_name="c", subcore_axis_name="s")
    out_type = [jax.ShapeDtypeStruct((N, 16), jnp.float32)] * nh
    scratch = [
        pltpu.VMEM((IDXS, kb, WIN), jnp.int32),
        pltpu.VMEM((IDXS, kb, WIN), jnp.int32),
        pltpu.VMEM((ROWS, kb, WIN, 16), jnp.float32),
        pltpu.VMEM_SHARED((N, 16), jnp.float32),
        pltpu.SemaphoreType.DMA,
        pltpu.SemaphoreType.DMA((kb,)),
        pltpu.SemaphoreType.DMA,
    ]
    if with_cnt:
        out_type.append(jax.ShapeDtypeStruct((N,), jnp.float32))
        scratch += [
            pltpu.VMEM((WIN,), jnp.float32),
            pltpu.VMEM_SHARED((N,), jnp.float32),
        ]

    def body(*refs):
        h_hbms = refs[:nh]
        ei_hbm, zrows_hbm, zflat_hbm = refs[nh:nh + 3]
        out_hbms = refs[nh + 3:nh + 3 + nh]
        rest = refs[nh + 3 + nh:]
        if with_cnt:
            (cnt_hbm, src_v, dst_v, rows_v, agg_sh, isem, gsem, ssem,
             ones_v, cnt_sh) = rest
        else:
            src_v, dst_v, rows_v, agg_sh, isem, gsem, ssem = rest
        c = lax.axis_index("c")
        s = lax.axis_index("s")

        def chunked(fn):
            @pl.when(s < 15)
            def _():
                fn(s * CCH, CCH)

            @pl.when(s == 15)
            def _():
                fn(15 * CCH, CREM)

        def zero_agg(off, sz):
            pltpu.sync_copy(zrows_hbm.at[pl.ds(off, sz)],
                            agg_sh.at[pl.ds(off, sz)])
        chunked(zero_agg)
        if with_cnt:
            for i in range(WIN // 16):
                ones_v[pl.ds(i * 16, 16)] = jnp.ones((16,), jnp.float32)

            @pl.when(c == 0)
            def _():
                def zero_cnt(off, sz):
                    pltpu.sync_copy(zflat_hbm.at[pl.ds(off, sz)],
                                    cnt_sh.at[pl.ds(off, sz)])
                chunked(zero_cnt)
        plsc.subcore_barrier()

        base = s * bpt + jnp.minimum(s, brem)
        nblk = jnp.where(s < brem, bpt + 1, bpt)

        def sweep(h_hbm, count_too):
            def issue_idx(g, islot):
                off = (base + g) * kb
                pltpu.async_copy(ei_hbm.at[0, pl.ds(off, kb)],
                                 src_v.at[islot], isem)
                pltpu.async_copy(ei_hbm.at[1, pl.ds(off, kb)],
                                 dst_v.at[islot], isem)

            def drain_block_scatters(rslot):
                for j in range(kb):
                    pltpu.make_async_copy(
                        zrows_hbm.at[pl.ds(0, WIN)], rows_v.at[rslot, j],
                        ssem).wait()
                    if count_too:
                        pltpu.make_async_copy(
                            zflat_hbm.at[pl.ds(0, WIN)], ones_v,
                            ssem).wait()

            issue_idx(0, 0)

            def step(g, carry):
                islot = lax.rem(g, IDXS)
                rslot = lax.rem(g, ROWS)
                pltpu.make_async_copy(ei_hbm.at[0, pl.ds(0, kb)],
                                      src_v.at[islot], isem).wait()
                pltpu.make_async_copy(ei_hbm.at[1, pl.ds(0, kb)],
                                      dst_v.at[islot], isem).wait()

                @pl.when(g >= ROWS)
                def _():
                    drain_block_scatters(rslot)

                @pl.when(g + 1 < nblk)
                def _():
                    issue_idx(g + 1, lax.rem(g + 1, IDXS))

                gds = [pltpu.async_copy(h_hbm.at[src_v.at[islot, j]],
                                        rows_v.at[rslot, j], gsem.at[j])
                       for j in range(kb)]
                for j in range(kb):
                    gds[j].wait()
                    pltpu.async_copy(rows_v.at[rslot, j],
                                     agg_sh.at[dst_v.at[islot, j]], ssem,
                                     add=True)
                    if count_too:
                        pltpu.async_copy(ones_v,
                                         cnt_sh.at[dst_v.at[islot, j]],
                                         ssem, add=True)
                return carry

            lax.fori_loop(0, nblk, step, 0)
            for r in range(ROWS):
                drain_block_scatters(r)

        def phase(pairs, core_cnt):
            for k, (h_hbm, out_hbm) in enumerate(pairs):
                if k > 0:
                    chunked(zero_agg)
                    plsc.subcore_barrier()
                do_cnt = core_cnt and k == 0
                sweep(h_hbm, do_cnt)
                plsc.subcore_barrier()

                def write(off, sz):
                    rows = pl.ds(off, sz)
                    pltpu.sync_copy(agg_sh.at[rows], out_hbm.at[rows])
                    if do_cnt:
                        pltpu.sync_copy(cnt_sh.at[rows], cnt_hbm.at[rows])
                chunked(write)

        @pl.when(c == 0)
        def _():
            phase(list(zip(h_hbms[:per_core], out_hbms[:per_core])),
                  with_cnt)

        @pl.when(c == 1)
        def _():
            phase(list(zip(h_hbms[per_core:], out_hbms[per_core:])),
                  False)

    fn = pl.kernel(
        body, out_type=out_type, mesh=mesh, scratch_types=scratch,
        compiler_params=pltpu.CompilerParams(use_tc_tiling_on_sc=False))
    return fn(*hs, ei, zrows, zflat)


def _pre_body(x_ref, W_ref, b_ref, ha_ref, hb_ref):
    h = jnp.maximum(jnp.dot(x_ref[...], W_ref[...],
                            preferred_element_type=jnp.float32)
                    + b_ref[...][None, :], 0.0)
    ha_ref[...] = h[:, :16]
    hb_ref[...] = h[:, 16:]


def _pre(x, W_pre, b_pre):
    return pl.pallas_call(
        _pre_body,
        grid=(NBLK,),
        in_specs=[
            pl.BlockSpec((ROWS_BLK, 5), lambda i: (i, 0)),
            pl.BlockSpec((5, 32), lambda i: (0, 0)),
            pl.BlockSpec((32,), lambda i: (0,)),
        ],
        out_specs=[
            pl.BlockSpec((ROWS_BLK, 16), lambda i: (i, 0)),
            pl.BlockSpec((ROWS_BLK, 16), lambda i: (i, 0)),
        ],
        out_shape=[jax.ShapeDtypeStruct((N, 16), jnp.float32)] * 2,
    )(x, W_pre, b_pre)


def _sage_update(agg, cnt2d, h, Wl, bl, Wr):
    recip = 1.0 / jnp.maximum(cnt2d, 1.0)
    out = (jnp.dot(agg * recip, Wl,
                   preferred_element_type=jnp.float32)
           + bl[None, :]
           + jnp.dot(h, Wr, preferred_element_type=jnp.float32))
    norm = jnp.sqrt(jnp.sum(out * out, axis=-1, keepdims=True))
    return jnp.maximum(out / jnp.maximum(norm, 1e-12), 0.0)


def _mid_body(aggL_ref, aggR_ref, cnt_ref, ha_ref, hb_ref,
              Wl_ref, bl_ref, Wr_ref, *out_refs):
    agg = jnp.concatenate([aggL_ref[...], aggR_ref[...]], axis=1)
    h = jnp.concatenate([ha_ref[...], hb_ref[...]], axis=1)
    h1 = _sage_update(agg, cnt_ref[...], h, Wl_ref[...], bl_ref[...],
                      Wr_ref[...])
    for k in range(4):
        out_refs[k][...] = h1[:, 16 * k:16 * (k + 1)]


def _mid(aggL, aggR, cnt, ha, hb, Wl1, bl1, Wr1):
    blk16 = pl.BlockSpec((ROWS_BLK, 16), lambda i: (i, 0))
    return pl.pallas_call(
        _mid_body,
        grid=(NBLK,),
        in_specs=[
            blk16, blk16,
            pl.BlockSpec((ROWS_BLK, 1), lambda i: (i, 0)),
            blk16, blk16,
            pl.BlockSpec((32, 64), lambda i: (0, 0)),
            pl.BlockSpec((64,), lambda i: (0,)),
            pl.BlockSpec((32, 64), lambda i: (0, 0)),
        ],
        out_specs=[blk16] * 4,
        out_shape=[jax.ShapeDtypeStruct((N, 16), jnp.float32)] * 4,
    )(aggL, aggR, cnt, ha, hb, Wl1, bl1, Wr1)


def _final_body(aggs_and_more, s_acc, c_acc):
    (aggA, aggB, aggC, aggD, cnt_ref, hA, hB, hC, hD,
     Wl_ref, bl_ref, Wr_ref, batch_ref,
     Wp1_ref, bp1_ref, Wp2_ref, bp2_ref, Wo_ref, bo_ref, out_ref) = \
        aggs_and_more
    i = pl.program_id(0)
    agg = jnp.concatenate([aggA[...], aggB[...], aggC[...], aggD[...]],
                          axis=1)
    h = jnp.concatenate([hA[...], hB[...], hC[...], hD[...]], axis=1)
    h2 = _sage_update(agg, cnt_ref[...], h, Wl_ref[...], bl_ref[...],
                      Wr_ref[...])
    onehot = (batch_ref[...]
              == lax.broadcasted_iota(jnp.int32, (ROWS_BLK, G), 1)
              ).astype(jnp.float32)
    contrib = lax.dot_general(onehot, h2, (((0,), (0,)), ((), ())),
                              preferred_element_type=jnp.float32,
                              precision=lax.Precision.HIGHEST)
    csum = jnp.sum(onehot, axis=0)

    @pl.when(i == 0)
    def _():
        s_acc[...] = contrib
        c_acc[...] = csum[None, :]

    @pl.when(i > 0)
    def _():
        s_acc[...] += contrib
        c_acc[...] += csum[None, :]

    @pl.when(i == NBLK - 1)
    def _():
        cvec = jnp.maximum(c_acc[...][0, :], 1.0)
        g = s_acc[...] / cvec[:, None]
        g = jnp.maximum(jnp.dot(g, Wp1_ref[...],
                                preferred_element_type=jnp.float32)
                        + bp1_ref[...][None, :], 0.0)
        g = jnp.maximum(jnp.dot(g, Wp2_ref[...],
                                preferred_element_type=jnp.float32)
                        + bp2_ref[...][None, :], 0.0)
        out_ref[...] = (jnp.dot(g, Wo_ref[...],
                                preferred_element_type=jnp.float32)
                        + bo_ref[...][None, :])


def _final(aggA, aggB, aggC, aggD, cnt, hA, hB, hC, hD, Wl2, bl2, Wr2,
           batch, Wp1, bp1, Wp2, bp2, Wo, bo):
    blk16 = pl.BlockSpec((ROWS_BLK, 16), lambda i: (i, 0))
    full = lambda *shape: pl.BlockSpec(shape, lambda i: (0,) * len(shape))
    return pl.pallas_call(
        lambda *refs: _final_body(refs[:-2], refs[-2], refs[-1]),
        grid=(NBLK,),
        in_specs=[
            blk16, blk16, blk16, blk16,
            pl.BlockSpec((ROWS_BLK, 1), lambda i: (i, 0)),
            blk16, blk16, blk16, blk16,
            full(64, 64), full(64), full(64, 64),
            pl.BlockSpec((ROWS_BLK, 1), lambda i: (i, 0)),
            full(64, 64), full(64), full(64, 16), full(16),
            full(16, 1), full(1),
        ],
        out_specs=full(G, 1),
        out_shape=jax.ShapeDtypeStruct((G, 1), jnp.float32),
        scratch_shapes=[
            pltpu.VMEM((G, 64), jnp.float32),
            pltpu.VMEM((1, G), jnp.float32),
        ],
    )(aggA, aggB, aggC, aggD, cnt, hA, hB, hC, hD, Wl2, bl2, Wr2,
      batch, Wp1, bp1, Wp2, bp2, Wo, bo)


def kernel(x, edge_index, batch, W_pre, b_pre, Wl1, bl1, Wr1,
           Wl2, bl2, Wr2, Wp1, bp1, Wp2, bp2, Wo, bo):
    zrows = jnp.zeros((N, 16), jnp.float32)
    zflat = jnp.zeros((N,), jnp.float32)
    ei3 = edge_index.reshape(2, NWIN, WIN)

    h0a, h0b = _pre(x, W_pre, b_pre)
    agg1L, agg1R, cnt = _sc_agg((h0a, h0b), ei3, zrows, zflat,
                                with_cnt=True)
    cnt2d = cnt.reshape(N, 1)
    h1a, h1b, h1c, h1d = _mid(agg1L, agg1R, cnt2d, h0a, h0b, Wl1, bl1, Wr1)
    agg2A, agg2B = _sc_agg((h1a, h1b), ei3, zrows, zflat, with_cnt=False)
    agg2C, agg2D = _sc_agg((h1c, h1d), ei3, zrows, zflat, with_cnt=False)
    out = _final(agg2A, agg2B, agg2C, agg2D, cnt2d, h1a, h1b, h1c, h1d,
                 Wl2, bl2, Wr2, batch.reshape(N, 1), Wp1, bp1, Wp2, bp2,
                 Wo, bo)
    return jnp.squeeze(out, -1)


# speedup vs baseline: 1.0622x; 1.0176x over previous
import functools

import jax
import jax.numpy as jnp
from jax import lax
from jax.experimental import pallas as pl
from jax.experimental.pallas import tpu as pltpu
from jax.experimental.pallas import tpu_sc as plsc

N = 100000
E = 3200000
G = 128
NC, NS = 2, 16
WIN = 128
NWIN = E // WIN
IDXS = 3
ROWS = 2
CCH = 6256
CREM = N - 15 * CCH

ROWS_BLK = 4000
NBLK = N // ROWS_BLK


def _sc_agg(hs, ei, zrows, zflat, with_cnt):
    nh = len(hs)
    per_core = nh // 2
    kb = 5
    nblock = NWIN // kb
    bpt = nblock // NS
    brem = nblock - bpt * NS
    mesh = plsc.VectorSubcoreMesh(core_axis_name="c", subcore_axis_name="s")
    out_type = [jax.ShapeDtypeStruct((N, 16), jnp.float32)] * nh
    scratch = [
        pltpu.VMEM((IDXS, kb, WIN), jnp.int32),
        pltpu.VMEM((IDXS, kb, WIN), jnp.int32),
        pltpu.VMEM((ROWS, kb, WIN, 16), jnp.float32),
        pltpu.VMEM_SHARED((N, 16), jnp.float32),
        pltpu.SemaphoreType.DMA,
        pltpu.SemaphoreType.DMA((kb,)),
        pltpu.SemaphoreType.DMA,
    ]
    if with_cnt:
        out_type.append(jax.ShapeDtypeStruct((N,), jnp.float32))
        scratch += [
            pltpu.VMEM((WIN,), jnp.float32),
            pltpu.VMEM_SHARED((N,), jnp.float32),
        ]

    def body(*refs):
        h_hbms = refs[:nh]
        ei_hbm, zrows_hbm, zflat_hbm = refs[nh:nh + 3]
        out_hbms = refs[nh + 3:nh + 3 + nh]
        rest = refs[nh + 3 + nh:]
        if with_cnt:
            (cnt_hbm, src_v, dst_v, rows_v, agg_sh, isem, gsem, ssem,
             ones_v, cnt_sh) = rest
        else:
            src_v, dst_v, rows_v, agg_sh, isem, gsem, ssem = rest
        c = lax.axis_index("c")
        s = lax.axis_index("s")

        def chunked(fn):
            @pl.when(s < 15)
            def _():
                fn(s * CCH, CCH)

            @pl.when(s == 15)
            def _():
                fn(15 * CCH, CREM)

        def zero_agg(off, sz):
            pltpu.sync_copy(zrows_hbm.at[pl.ds(off, sz)],
                            agg_sh.at[pl.ds(off, sz)])
        chunked(zero_agg)
        if with_cnt:
            for i in range(WIN // 16):
                ones_v[pl.ds(i * 16, 16)] = jnp.ones((16,), jnp.float32)

            @pl.when(c == 0)
            def _():
                def zero_cnt(off, sz):
                    pltpu.sync_copy(zflat_hbm.at[pl.ds(off, sz)],
                                    cnt_sh.at[pl.ds(off, sz)])
                chunked(zero_cnt)
        plsc.subcore_barrier()

        base = s * bpt + jnp.minimum(s, brem)
        nblk = jnp.where(s < brem, bpt + 1, bpt)

        def sweep(h_hbm, count_too):
            def issue_idx(g, islot):
                off = (base + g) * kb
                pltpu.async_copy(ei_hbm.at[0, pl.ds(off, kb)],
                                 src_v.at[islot], isem)
                pltpu.async_copy(ei_hbm.at[1, pl.ds(off, kb)],
                                 dst_v.at[islot], isem)

            def drain_block_scatters(rslot):
                for j in range(kb):
                    pltpu.make_async_copy(
                        zrows_hbm.at[pl.ds(0, WIN)], rows_v.at[rslot, j],
                        ssem).wait()
                    if count_too:
                        pltpu.make_async_copy(
                            zflat_hbm.at[pl.ds(0, WIN)], ones_v,
                            ssem).wait()

            issue_idx(0, 0)

            def step(g, carry):
                islot = lax.rem(g, IDXS)
                rslot = lax.rem(g, ROWS)
                pltpu.make_async_copy(ei_hbm.at[0, pl.ds(0, kb)],
                                      src_v.at[islot], isem).wait()
                pltpu.make_async_copy(ei_hbm.at[1, pl.ds(0, kb)],
                                      dst_v.at[islot], isem).wait()

                @pl.when(g >= ROWS)
                def _():
                    drain_block_scatters(rslot)

                @pl.when(g + 1 < nblk)
                def _():
                    issue_idx(g + 1, lax.rem(g + 1, IDXS))

                gds = [pltpu.async_copy(h_hbm.at[src_v.at[islot, j]],
                                        rows_v.at[rslot, j], gsem.at[j])
                       for j in range(kb)]
                for j in range(kb):
                    gds[j].wait()
                    pltpu.async_copy(rows_v.at[rslot, j],
                                     agg_sh.at[dst_v.at[islot, j]], ssem,
                                     add=True)
                    if count_too:
                        pltpu.async_copy(ones_v,
                                         cnt_sh.at[dst_v.at[islot, j]],
                                         ssem, add=True)
                return carry

            lax.fori_loop(0, nblk, step, 0)
            for r in range(ROWS):
                drain_block_scatters(r)

        def phase(pairs, core_cnt):
            for k, (h_hbm, out_hbm) in enumerate(pairs):
                if k > 0:
                    chunked(zero_agg)
                    plsc.subcore_barrier()
                do_cnt = core_cnt and k == 0
                sweep(h_hbm, do_cnt)
                plsc.subcore_barrier()

                def write(off, sz):
                    rows = pl.ds(off, sz)
                    pltpu.sync_copy(agg_sh.at[rows], out_hbm.at[rows])
                    if do_cnt:
                        pltpu.sync_copy(cnt_sh.at[rows], cnt_hbm.at[rows])
                chunked(write)

        @pl.when(c == 0)
        def _():
            phase(list(zip(h_hbms[:per_core], out_hbms[:per_core])),
                  with_cnt)

        @pl.when(c == 1)
        def _():
            phase(list(zip(h_hbms[per_core:], out_hbms[per_core:])),
                  False)

    fn = pl.kernel(
        body, out_type=out_type, mesh=mesh, scratch_types=scratch,
        compiler_params=pltpu.CompilerParams(use_tc_tiling_on_sc=False))
    return fn(*hs, ei, zrows, zflat)


def _pre_body(x_ref, W_ref, b_ref, ha_ref, hb_ref):
    h = jnp.maximum(jnp.dot(x_ref[...], W_ref[...],
                            preferred_element_type=jnp.float32)
                    + b_ref[...][None, :], 0.0)
    ha_ref[...] = h[:, :16]
    hb_ref[...] = h[:, 16:]


def _pre(x, W_pre, b_pre):
    return pl.pallas_call(
        _pre_body,
        grid=(NBLK,),
        in_specs=[
            pl.BlockSpec((ROWS_BLK, 5), lambda i: (i, 0)),
            pl.BlockSpec((5, 32), lambda i: (0, 0)),
            pl.BlockSpec((32,), lambda i: (0,)),
        ],
        out_specs=[
            pl.BlockSpec((ROWS_BLK, 16), lambda i: (i, 0)),
            pl.BlockSpec((ROWS_BLK, 16), lambda i: (i, 0)),
        ],
        out_shape=[jax.ShapeDtypeStruct((N, 16), jnp.float32)] * 2,
    )(x, W_pre, b_pre)


def _sage_update(agg, cnt2d, h, Wl, bl, Wr):
    recip = 1.0 / jnp.maximum(cnt2d, 1.0)
    out = (jnp.dot(agg * recip, Wl,
                   preferred_element_type=jnp.float32)
           + bl[None, :]
           + jnp.dot(h, Wr, preferred_element_type=jnp.float32))
    norm = jnp.sqrt(jnp.sum(out * out, axis=-1, keepdims=True))
    return jnp.maximum(out / jnp.maximum(norm, 1e-12), 0.0)


def _mid_body(aggL_ref, aggR_ref, cnt_ref, ha_ref, hb_ref,
              Wl_ref, bl_ref, Wr_ref, *out_refs):
    agg = jnp.concatenate([aggL_ref[...], aggR_ref[...]], axis=1)
    h = jnp.concatenate([ha_ref[...], hb_ref[...]], axis=1)
    h1 = _sage_update(agg, cnt_ref[...], h, Wl_ref[...], bl_ref[...],
                      Wr_ref[...])
    for k in range(4):
        out_refs[k][...] = h1[:, 16 * k:16 * (k + 1)]


def _mid(aggL, aggR, cnt, ha, hb, Wl1, bl1, Wr1):
    blk16 = pl.BlockSpec((ROWS_BLK, 16), lambda i: (i, 0))
    return pl.pallas_call(
        _mid_body,
        grid=(NBLK,),
        in_specs=[
            blk16, blk16,
            pl.BlockSpec((ROWS_BLK, 1), lambda i: (i, 0)),
            blk16, blk16,
            pl.BlockSpec((32, 64), lambda i: (0, 0)),
            pl.BlockSpec((64,), lambda i: (0,)),
            pl.BlockSpec((32, 64), lambda i: (0, 0)),
        ],
        out_specs=[blk16] * 4,
        out_shape=[jax.ShapeDtypeStruct((N, 16), jnp.float32)] * 4,
    )(aggL, aggR, cnt, ha, hb, Wl1, bl1, Wr1)


def _final_body(aggs_and_more, s_acc, c_acc):
    (aggA, aggB, aggC, aggD, cnt_ref, hA, hB, hC, hD,
     Wl_ref, bl_ref, Wr_ref, batch_ref,
     Wp1_ref, bp1_ref, Wp2_ref, bp2_ref, Wo_ref, bo_ref, out_ref) = \
        aggs_and_more
    i = pl.program_id(0)
    agg = jnp.concatenate([aggA[...], aggB[...], aggC[...], aggD[...]],
                          axis=1)
    h = jnp.concatenate([hA[...], hB[...], hC[...], hD[...]], axis=1)
    h2 = _sage_update(agg, cnt_ref[...], h, Wl_ref[...], bl_ref[...],
                      Wr_ref[...])
    onehot = (batch_ref[...]
              == lax.broadcasted_iota(jnp.int32, (ROWS_BLK, G), 1)
              ).astype(jnp.float32)
    contrib = lax.dot_general(onehot, h2, (((0,), (0,)), ((), ())),
                              preferred_element_type=jnp.float32,
                              precision=lax.Precision.HIGHEST)
    csum = jnp.sum(onehot, axis=0)

    @pl.when(i == 0)
    def _():
        s_acc[...] = contrib
        c_acc[...] = csum[None, :]

    @pl.when(i > 0)
    def _():
        s_acc[...] += contrib
        c_acc[...] += csum[None, :]

    @pl.when(i == NBLK - 1)
    def _():
        cvec = jnp.maximum(c_acc[...][0, :], 1.0)
        g = s_acc[...] / cvec[:, None]
        g = jnp.maximum(jnp.dot(g, Wp1_ref[...],
                                preferred_element_type=jnp.float32)
                        + bp1_ref[...][None, :], 0.0)
        g = jnp.maximum(jnp.dot(g, Wp2_ref[...],
                                preferred_element_type=jnp.float32)
                        + bp2_ref[...][None, :], 0.0)
        out_ref[...] = (jnp.dot(g, Wo_ref[...],
                                preferred_element_type=jnp.float32)
                        + bo_ref[...][None, :])


def _final(aggA, aggB, aggC, aggD, cnt, hA, hB, hC, hD, Wl2, bl2, Wr2,
           batch, Wp1, bp1, Wp2, bp2, Wo, bo):
    blk16 = pl.BlockSpec((ROWS_BLK, 16), lambda i: (i, 0))
    full = lambda *shape: pl.BlockSpec(shape, lambda i: (0,) * len(shape))
    return pl.pallas_call(
        lambda *refs: _final_body(refs[:-2], refs[-2], refs[-1]),
        grid=(NBLK,),
        in_specs=[
            blk16, blk16, blk16, blk16,
            pl.BlockSpec((ROWS_BLK, 1), lambda i: (i, 0)),
            blk16, blk16, blk16, blk16,
            full(64, 64), full(64), full(64, 64),
            pl.BlockSpec((ROWS_BLK, 1), lambda i: (i, 0)),
            full(64, 64), full(64), full(64, 16), full(16),
            full(16, 1), full(1),
        ],
        out_specs=full(G, 1),
        out_shape=jax.ShapeDtypeStruct((G, 1), jnp.float32),
        scratch_shapes=[
            pltpu.VMEM((G, 64), jnp.float32),
            pltpu.VMEM((1, G), jnp.float32),
        ],
    )(aggA, aggB, aggC, aggD, cnt, hA, hB, hC, hD, Wl2, bl2, Wr2,
      batch, Wp1, bp1, Wp2, bp2, Wo, bo)


def kernel(x, edge_index, batch, W_pre, b_pre, Wl1, bl1, Wr1,
           Wl2, bl2, Wr2, Wp1, bp1, Wp2, bp2, Wo, bo):
    zrows = jnp.zeros((N, 16), jnp.float32)
    zflat = jnp.zeros((N,), jnp.float32)
    ei3 = edge_index.reshape(2, NWIN, WIN)

    h0a, h0b = _pre(x, W_pre, b_pre)
    agg1L, agg1R, cnt = _sc_agg((h0a, h0b), ei3, zrows, zflat,
                                with_cnt=True)
    cnt2d = cnt.reshape(N, 1)
    h1a, h1b, h1c, h1d = _mid(agg1L, agg1R, cnt2d, h0a, h0b, Wl1, bl1, Wr1)
    agg2A, agg2B = _sc_agg((h1a, h1b), ei3, zrows, zflat, with_cnt=False)
    agg2C, agg2D = _sc_agg((h1c, h1d), ei3, zrows, zflat, with_cnt=False)
    out = _final(agg2A, agg2B, agg2C, agg2D, cnt2d, h1a, h1b, h1c, h1d,
                 Wl2, bl2, Wr2, batch.reshape(N, 1), Wp1, bp1, Wp2, bp2,
                 Wo, bo)
    return jnp.squeeze(out, -1)
